# baseline (speedup 1.0000x reference)
import jax
import jax.numpy as jnp
from jax import lax
from jax.experimental import pallas as pl
from jax.experimental.pallas import tpu as pltpu
from jax.experimental.pallas import tpu_sc as plsc

_N = 50000
_H = 256
_S = 64
_A = 16
_L = 16
_NC = _H // _L
_NW = 32
_RB = 80
_NTC = 10000
_RT = 1000
_NBT = _NTC // _RT
_NSC = _N - _NTC
_NB = _NSC // _RB
_IT = (_NB + _NW - 1) // _NW

_NEG = float("-inf")


def _i32(x):
    return jnp.asarray(x, jnp.int32)


def _neg_vec():
    return jnp.full((_L,), _NEG, jnp.float32)


def _sc_body(z_hbm, batch_hbm, out_hbm, zbuf, bbuf, sem0, sem1, acc):
    wid = lax.axis_index("s") * _i32(2) + lax.axis_index("c")
    sems = (sem0, sem1)
    _BP = _RB + _L

    def init_body(i, carry):
        for c in range(_NC):
            acc[i, pl.ds(c * _L, _L)] = _neg_vec()
        return carry
    lax.fori_loop(_i32(0), _i32(_S), init_body, _i32(0))

    start_blk = wid * _i32(_IT)
    nblk = jnp.clip(_i32(_NB) - start_blk, _i32(0), _i32(_IT))

    def start_dma(it, par):
        base = (start_blk + it) * _i32(_RB)
        pltpu.make_async_copy(
            z_hbm.at[pl.ds(base, _RB)],
            zbuf.at[pl.ds(par * _RB, _RB)], sems[par]).start()
        pltpu.make_async_copy(
            batch_hbm.at[pl.ds(base, _RB)],
            bbuf.at[pl.ds(par * _BP, _RB)], sems[par]).start()

    def wait_dma(par):
        pltpu.make_async_copy(
            z_hbm.at[pl.ds(0, _RB)],
            zbuf.at[pl.ds(par * _RB, _RB)], sems[par]).wait()
        pltpu.make_async_copy(
            batch_hbm.at[pl.ds(0, _RB)],
            bbuf.at[pl.ds(par * _BP, _RB)], sems[par]).wait()

    @pl.when(nblk > _i32(0))
    def _prime():
        start_dma(_i32(0), 0)

    def blk_body(it, carry):
        par_bit = lax.bitwise_and(it, _i32(1))

        @pl.when(par_bit == _i32(0))
        def _():
            wait_dma(0)

        @pl.when(par_bit == _i32(1))
        def _():
            wait_dma(1)

        @pl.when(jnp.logical_and(it + _i32(1) < nblk, par_bit == _i32(0)))
        def _():
            start_dma(it + _i32(1), 1)

        @pl.when(jnp.logical_and(it + _i32(1) < nblk, par_bit == _i32(1)))
        def _():
            start_dma(it + _i32(1), 0)

        zoff = par_bit * _i32(_RB)
        boff = par_bit * _i32(_BP)

        def grp_body(g, c2):
            gbase = zoff + g * _i32(_L)
            bbase = boff + g * _i32(_L)
            bvec = bbuf[pl.ds(bbase, _L)]
            s0 = bvec[0]
            s15 = bvec[_L - 1]

            @pl.when(s0 == s15)
            def _fast():
                for c in range(_NC):
                    sl = pl.ds(c * _L, _L)
                    vals = [zbuf[gbase + _i32(j), sl] for j in range(_L)]
                    while len(vals) > 1:
                        nxt = [jnp.maximum(vals[i], vals[i + 1])
                               for i in range(0, len(vals) - 1, 2)]
                        if len(vals) % 2:
                            nxt.append(vals[-1])
                        vals = nxt
                    acc[s0, sl] = jnp.maximum(acc[s0, sl], vals[0])

            @pl.when(s0 != s15)
            def _slow():
                def row_body(j, c3):
                    bv = bbuf[pl.ds(bbase + j, _L)]
                    s = bv[0]
                    rz = gbase + j
                    for c in range(_NC):
                        sl = pl.ds(c * _L, _L)
                        acc[s, sl] = jnp.maximum(acc[s, sl], zbuf[rz, sl])
                    return c3
                lax.fori_loop(_i32(0), _i32(_L), row_body, _i32(0))

            return c2

        lax.fori_loop(_i32(0), _i32(_RB // _L), grp_body, _i32(0))
        return carry

    lax.fori_loop(_i32(0), nblk, blk_body, _i32(0))
    pltpu.sync_copy(acc, out_hbm.at[wid])


def _sc_pool(z, batch32):
    mesh = plsc.VectorSubcoreMesh(core_axis_name="c", subcore_axis_name="s")
    return pl.kernel(
        _sc_body,
        out_type=jax.ShapeDtypeStruct((_NW, _S, _H), jnp.float32),
        mesh=mesh,
        scratch_types=[
            pltpu.VMEM((2 * _RB, _H), jnp.float32),
            pltpu.VMEM((2 * (_RB + _L),), jnp.int32),
            pltpu.SemaphoreType.DMA,
            pltpu.SemaphoreType.DMA,
            pltpu.VMEM((_S, _H), jnp.float32),
        ],
    )(z, batch32)


def _zero():
    return jnp.zeros((), jnp.int32)


def _tc_pool_body(bi_ref, z_ref, out_ref, acc_ref):
    blk = pl.program_id(0)

    @pl.when(blk == 0)
    def _init():
        acc_ref[...] = jnp.full((_S, _H), _NEG, jnp.float32)

    zb = z_ref[...]
    bi = bi_ref[0]
    lo = jnp.min(bi)
    hi = jnp.max(bi)

    def seg_body(s, carry):
        mask = bi == s
        vals = jnp.where(mask, zb, _NEG)
        m = jnp.max(vals, axis=0, keepdims=True)
        cur = acc_ref[pl.ds(s, 1), :]
        acc_ref[pl.ds(s, 1), :] = jnp.maximum(cur, m)
        return carry

    lax.fori_loop(lo, hi + _i32(1), seg_body, _i32(0))

    @pl.when(blk == _NBT - 1)
    def _out():
        out_ref[...] = acc_ref[...]


def _tc_pool(z, batch32):
    off = _NSC // _RT
    return pl.pallas_call(
        _tc_pool_body,
        grid=(_NBT,),
        in_specs=[
            pl.BlockSpec((1, _RT, 1), lambda i: (i + off, _zero(), _zero())),
            pl.BlockSpec((_RT, _H), lambda i: (i + off, _zero())),
        ],
        out_specs=pl.BlockSpec((_S, _H), lambda i: (_zero(), _zero())),
        out_shape=jax.ShapeDtypeStruct((_S, _H), jnp.float32),
        scratch_shapes=[pltpu.VMEM((_S, _H), jnp.float32)],
    )(batch32.reshape(_N // _RT, _RT, 1), z)


def _tc_merge_body(p_ref, t_ref, w_ref, b_ref, out_ref):
    pooled = jnp.maximum(jnp.max(p_ref[...], axis=0), t_ref[...])
    out = lax.dot_general(
        pooled, w_ref[...], (((1,), (1,)), ((), ())),
        preferred_element_type=jnp.float32)
    out_ref[...] = out + b_ref[...]


def _tc_merge(partials, tcpart, W, b2):
    return pl.pallas_call(
        _tc_merge_body,
        out_shape=jax.ShapeDtypeStruct((_S, _A), jnp.float32),
    )(partials, tcpart, W, b2)


def kernel(z, edge_index, batch, W, b):
    batch32 = batch.astype(jnp.int32)
    b2 = b.reshape(1, _A)
    partials = _sc_pool(z, batch32)
    tcpart = _tc_pool(z, batch32)
    return _tc_merge(partials, tcpart, W, b2)


# speedup vs baseline: 1.2134x; 1.2134x over previous
import jax
import jax.numpy as jnp
from jax import lax
from jax.experimental import pallas as pl
from jax.experimental.pallas import tpu as pltpu
from jax.experimental.pallas import tpu_sc as plsc

_N = 50000
_H = 256
_S = 64
_A = 16
_L = 16
_NC = _H // _L
_NW = 32
_RB = 80
_NB = _N // _RB
_IT = (_NB + _NW - 1) // _NW

_NEG = float("-inf")


def _i32(x):
    return jnp.asarray(x, jnp.int32)


def _neg_vec():
    return jnp.full((_L,), _NEG, jnp.float32)


def _sc_body(z_hbm, batch_hbm, out_hbm, zbuf, bbuf, sem0, sem1, acc):
    wid = lax.axis_index("s") * _i32(2) + lax.axis_index("c")
    sems = (sem0, sem1)
    _BP = _RB + _L

    def init_body(i, carry):
        for c in range(_NC):
            acc[i, pl.ds(c * _L, _L)] = _neg_vec()
        return carry
    lax.fori_loop(_i32(0), _i32(_S), init_body, _i32(0))

    start_blk = wid * _i32(_IT)
    nblk = jnp.clip(_i32(_NB) - start_blk, _i32(0), _i32(_IT))

    def start_dma(it, par):
        base = (start_blk + it) * _i32(_RB)
        pltpu.make_async_copy(
            z_hbm.at[pl.ds(base, _RB)],
            zbuf.at[pl.ds(par * _RB, _RB)], sems[par]).start()
        pltpu.make_async_copy(
            batch_hbm.at[pl.ds(base, _RB)],
            bbuf.at[pl.ds(par * _BP, _RB)], sems[par]).start()

    def wait_dma(par):
        pltpu.make_async_copy(
            z_hbm.at[pl.ds(0, _RB)],
            zbuf.at[pl.ds(par * _RB, _RB)], sems[par]).wait()
        pltpu.make_async_copy(
            batch_hbm.at[pl.ds(0, _RB)],
            bbuf.at[pl.ds(par * _BP, _RB)], sems[par]).wait()

    @pl.when(nblk > _i32(0))
    def _prime():
        start_dma(_i32(0), 0)

    def blk_body(it, carry):
        par_bit = lax.bitwise_and(it, _i32(1))

        @pl.when(par_bit == _i32(0))
        def _():
            wait_dma(0)

        @pl.when(par_bit == _i32(1))
        def _():
            wait_dma(1)

        @pl.when(jnp.logical_and(it + _i32(1) < nblk, par_bit == _i32(0)))
        def _():
            start_dma(it + _i32(1), 1)

        @pl.when(jnp.logical_and(it + _i32(1) < nblk, par_bit == _i32(1)))
        def _():
            start_dma(it + _i32(1), 0)

        zoff = par_bit * _i32(_RB)
        boff = par_bit * _i32(_BP)

        def fast_group(gbase, s0):
            for c in range(_NC):
                sl = pl.ds(c * _L, _L)
                vals = [zbuf[gbase + _i32(j), sl] for j in range(_L)]
                while len(vals) > 1:
                    nxt = [jnp.maximum(vals[i], vals[i + 1])
                           for i in range(0, len(vals) - 1, 2)]
                    if len(vals) % 2:
                        nxt.append(vals[-1])
                    vals = nxt
                acc[s0, sl] = jnp.maximum(acc[s0, sl], vals[0])

        b0 = bbuf[pl.ds(boff, _L)][0]
        blast = bbuf[pl.ds(boff + _i32(_RB - _L), _L)][_L - 1]

        @pl.when(b0 == blast)
        def _uniform_block():
            def ugrp_body(g, c2):
                fast_group(zoff + g * _i32(_L), b0)
                return c2
            lax.fori_loop(_i32(0), _i32(_RB // _L), ugrp_body, _i32(0))

        @pl.when(b0 != blast)
        def _mixed_block():
            def grp_body(g, c2):
                gbase = zoff + g * _i32(_L)
                bbase = boff + g * _i32(_L)
                bvec = bbuf[pl.ds(bbase, _L)]
                s0 = bvec[0]
                s15 = bvec[_L - 1]

                @pl.when(s0 == s15)
                def _fast():
                    fast_group(gbase, s0)

                @pl.when(s0 != s15)
                def _slow():
                    def row_body(j, c3):
                        bv = bbuf[pl.ds(bbase + j, _L)]
                        s = bv[0]
                        rz = gbase + j
                        for c in range(_NC):
                            sl = pl.ds(c * _L, _L)
                            acc[s, sl] = jnp.maximum(acc[s, sl], zbuf[rz, sl])
                        return c3
                    lax.fori_loop(_i32(0), _i32(_L), row_body, _i32(0))

                return c2

            lax.fori_loop(_i32(0), _i32(_RB // _L), grp_body, _i32(0))
        return carry

    lax.fori_loop(_i32(0), nblk, blk_body, _i32(0))
    pltpu.sync_copy(acc, out_hbm.at[wid])


def _sc_pool(z, batch32):
    mesh = plsc.VectorSubcoreMesh(core_axis_name="c", subcore_axis_name="s")
    return pl.kernel(
        _sc_body,
        out_type=jax.ShapeDtypeStruct((_NW, _S, _H), jnp.float32),
        mesh=mesh,
        scratch_types=[
            pltpu.VMEM((2 * _RB, _H), jnp.float32),
            pltpu.VMEM((2 * (_RB + _L),), jnp.int32),
            pltpu.SemaphoreType.DMA,
            pltpu.SemaphoreType.DMA,
            pltpu.VMEM((_S, _H), jnp.float32),
        ],
    )(z, batch32)


def _tc_merge_body(p_ref, w_ref, b_ref, out_ref):
    pooled = jnp.max(p_ref[...], axis=0)
    out = lax.dot_general(
        pooled, w_ref[...], (((1,), (1,)), ((), ())),
        preferred_element_type=jnp.float32)
    out_ref[...] = out + b_ref[...]


def _tc_merge(partials, W, b2):
    return pl.pallas_call(
        _tc_merge_body,
        out_shape=jax.ShapeDtypeStruct((_S, _A), jnp.float32),
    )(partials, W, b2)


def kernel(z, edge_index, batch, W, b):
    batch32 = batch.astype(jnp.int32)
    b2 = b.reshape(1, _A)
    partials = _sc_pool(z, batch32)
    return _tc_merge(partials, W, b2)


# speedup vs baseline: 1.2429x; 1.0243x over previous
import jax
import jax.numpy as jnp
from jax import lax
from jax.experimental import pallas as pl
from jax.experimental.pallas import tpu as pltpu
from jax.experimental.pallas import tpu_sc as plsc

_N = 50000
_H = 256
_S = 64
_A = 16
_L = 16
_NC = _H // _L
_NW = 32
_RB = 80
_NB = _N // _RB
_IT = (_NB + _NW - 1) // _NW

_NEG = float("-inf")


def _i32(x):
    return jnp.asarray(x, jnp.int32)


def _neg_vec():
    return jnp.full((_L,), _NEG, jnp.float32)


def _sc_body(z_hbm, batch_hbm, out_hbm, zbuf, bbuf, sem0, sem1, acc):
    wid = lax.axis_index("s") * _i32(2) + lax.axis_index("c")
    sems = (sem0, sem1)
    _BP = _RB + _L

    start_blk = wid * _i32(_IT)
    nblk = jnp.clip(_i32(_NB) - start_blk, _i32(0), _i32(_IT))

    def start_dma(it, par):
        base = (start_blk + it) * _i32(_RB)
        pltpu.make_async_copy(
            z_hbm.at[pl.ds(base, _RB)],
            zbuf.at[pl.ds(par * _RB, _RB)], sems[par]).start()
        pltpu.make_async_copy(
            batch_hbm.at[pl.ds(base, _RB)],
            bbuf.at[pl.ds(par * _BP, _RB)], sems[par]).start()

    def wait_dma(par):
        pltpu.make_async_copy(
            z_hbm.at[pl.ds(0, _RB)],
            zbuf.at[pl.ds(par * _RB, _RB)], sems[par]).wait()
        pltpu.make_async_copy(
            batch_hbm.at[pl.ds(0, _RB)],
            bbuf.at[pl.ds(par * _BP, _RB)], sems[par]).wait()

    @pl.when(nblk > _i32(0))
    def _prime():
        start_dma(_i32(0), 0)

    def init_body(i, carry):
        for c in range(_NC):
            acc[i, pl.ds(c * _L, _L)] = _neg_vec()
        return carry
    lax.fori_loop(_i32(0), _i32(_S), init_body, _i32(0))

    def blk_body(it, carry):
        par_bit = lax.bitwise_and(it, _i32(1))

        @pl.when(par_bit == _i32(0))
        def _():
            wait_dma(0)

        @pl.when(par_bit == _i32(1))
        def _():
            wait_dma(1)

        @pl.when(jnp.logical_and(it + _i32(1) < nblk, par_bit == _i32(0)))
        def _():
            start_dma(it + _i32(1), 1)

        @pl.when(jnp.logical_and(it + _i32(1) < nblk, par_bit == _i32(1)))
        def _():
            start_dma(it + _i32(1), 0)

        zoff = par_bit * _i32(_RB)
        boff = par_bit * _i32(_BP)

        def grp_body(g, c2):
            gbase = zoff + g * _i32(_L)
            bbase = boff + g * _i32(_L)
            bvec = bbuf[pl.ds(bbase, _L)]
            s0 = bvec[0]
            s15 = bvec[_L - 1]

            @pl.when(s0 == s15)
            def _fast():
                for c in range(_NC):
                    sl = pl.ds(c * _L, _L)
                    vals = [zbuf[gbase + _i32(j), sl] for j in range(_L)]
                    while len(vals) > 1:
                        nxt = [jnp.maximum(vals[i], vals[i + 1])
                               for i in range(0, len(vals) - 1, 2)]
                        if len(vals) % 2:
                            nxt.append(vals[-1])
                        vals = nxt
                    acc[s0, sl] = jnp.maximum(acc[s0, sl], vals[0])

            @pl.when(s0 != s15)
            def _slow():
                def row_body(j, c3):
                    bv = bbuf[pl.ds(bbase + j, _L)]
                    s = bv[0]
                    rz = gbase + j
                    for c in range(_NC):
                        sl = pl.ds(c * _L, _L)
                        acc[s, sl] = jnp.maximum(acc[s, sl], zbuf[rz, sl])
                    return c3
                lax.fori_loop(_i32(0), _i32(_L), row_body, _i32(0))

            return c2

        lax.fori_loop(_i32(0), _i32(_RB // _L), grp_body, _i32(0))
        return carry

    lax.fori_loop(_i32(0), nblk, blk_body, _i32(0))
    pltpu.sync_copy(acc, out_hbm.at[wid])


def _sc_pool(z, batch32):
    mesh = plsc.VectorSubcoreMesh(core_axis_name="c", subcore_axis_name="s")
    return pl.kernel(
        _sc_body,
        out_type=jax.ShapeDtypeStruct((_NW, _S, _H), jnp.float32),
        mesh=mesh,
        scratch_types=[
            pltpu.VMEM((2 * _RB, _H), jnp.float32),
            pltpu.VMEM((2 * (_RB + _L),), jnp.int32),
            pltpu.SemaphoreType.DMA,
            pltpu.SemaphoreType.DMA,
            pltpu.VMEM((_S, _H), jnp.float32),
        ],
    )(z, batch32)


def _tc_merge_body(p_ref, w_ref, b_ref, out_ref):
    pooled = jnp.max(p_ref[...], axis=0)
    out = lax.dot_general(
        pooled, w_ref[...], (((1,), (1,)), ((), ())),
        preferred_element_type=jnp.float32)
    out_ref[...] = out + b_ref[...]


def _tc_merge(partials, W, b2):
    return pl.pallas_call(
        _tc_merge_body,
        out_shape=jax.ShapeDtypeStruct((_S, _A), jnp.float32),
    )(partials, W, b2)


def kernel(z, edge_index, batch, W, b):
    batch32 = batch.astype(jnp.int32)
    b2 = b.reshape(1, _A)
    partials = _sc_pool(z, batch32)
    return _tc_merge(partials, W, b2)
